# baseline (speedup 1.0000x reference)
import jax
import jax.numpy as jnp
from jax.experimental import pallas as pl
from jax.experimental.pallas import tpu as pltpu

_C = 512


def kernel(tokens, W_pos):
    batch = tokens.shape[0]
    seq = tokens.shape[1]
    d = W_pos.shape[1]
    nch = seq // _C

    def body(w_hbm, out_hbm, buf, in_sem, out_sem):
        def make_in(i):
            return pltpu.make_async_copy(
                w_hbm.at[pl.ds(i * _C, _C)], buf.at[pl.ds(i * _C, _C)], in_sem
            )

        in_copies = [make_in(i) for i in range(nch)]
        in_copies[0].start()
        if nch > 1:
            in_copies[1].start()
        out_copies = []
        for i in range(nch):
            in_copies[i].wait()
            if i + 2 < nch:
                in_copies[i + 2].start()
            for b in range(batch):
                cc = pltpu.make_async_copy(
                    buf.at[pl.ds(i * _C, _C)],
                    out_hbm.at[b, pl.ds(i * _C, _C)],
                    out_sem,
                )
                cc.start()
                out_copies.append(cc)
        for c in out_copies:
            c.wait()

    out = pl.pallas_call(
        body,
        in_specs=[pl.BlockSpec(memory_space=pltpu.MemorySpace.HBM)],
        out_specs=pl.BlockSpec(memory_space=pltpu.MemorySpace.HBM),
        out_shape=jax.ShapeDtypeStruct((batch, seq, d), W_pos.dtype),
        scratch_shapes=[
            pltpu.VMEM((seq, d), W_pos.dtype),
            pltpu.SemaphoreType.DMA,
            pltpu.SemaphoreType.DMA,
        ],
    )(W_pos)
    return out


# speedup vs baseline: 1.0439x; 1.0439x over previous
import jax
import jax.numpy as jnp
from jax.experimental import pallas as pl
from jax.experimental.pallas import tpu as pltpu

_C = 1024


def kernel(tokens, W_pos):
    batch = tokens.shape[0]
    seq = tokens.shape[1]
    d = W_pos.shape[1]
    nch = seq // _C

    def body(w_hbm, out_hbm, buf, in_sem, out_sem):
        def make_in(i):
            return pltpu.make_async_copy(
                w_hbm.at[pl.ds(i * _C, _C)], buf.at[pl.ds(i * _C, _C)], in_sem
            )

        in_copies = [make_in(i) for i in range(nch)]
        for c in in_copies:
            c.start()
        out_copies = []
        for i in range(nch):
            in_copies[i].wait()
            for b in range(batch):
                cc = pltpu.make_async_copy(
                    buf.at[pl.ds(i * _C, _C)],
                    out_hbm.at[b, pl.ds(i * _C, _C)],
                    out_sem,
                )
                cc.start()
                out_copies.append(cc)
        for c in out_copies:
            c.wait()

    out = pl.pallas_call(
        body,
        in_specs=[pl.BlockSpec(memory_space=pltpu.MemorySpace.HBM)],
        out_specs=pl.BlockSpec(memory_space=pltpu.MemorySpace.HBM),
        out_shape=jax.ShapeDtypeStruct((batch, seq, d), W_pos.dtype),
        scratch_shapes=[
            pltpu.VMEM((seq, d), W_pos.dtype),
            pltpu.SemaphoreType.DMA,
            pltpu.SemaphoreType.DMA,
        ],
    )(W_pos)
    return out
